# Initial kernel scaffold; baseline (speedup 1.0000x reference)
import jax
import jax.numpy as jnp
from jax.experimental import pallas as pl


def kernel(x, enc0_w1, enc0_b1, enc0_w2, enc0_b2, enc0_wd, enc0_bd, enc1_w1, enc1_b1, enc1_w2, enc1_b2, enc2_w1, enc2_b1, enc2_w2, enc2_b2, enc_out_w, enc_out_b, dec0_w1, dec0_b1, dec0_w2, dec0_b2, dec0_wd, dec0_bd, dec1_w1, dec1_b1, dec1_w2, dec1_b2, dec2_w1, dec2_b1, dec2_w2, dec2_b2, dec_out_w, dec_out_b):
    raise NotImplementedError("write your pallas kernel here")


# speedup vs baseline: 1.1327x; 1.1327x over previous
import math

import jax
import jax.numpy as jnp
from jax.experimental import pallas as pl
from jax.experimental.pallas import tpu as pltpu

_SCALE = 1.0 / math.sqrt(1.0 + 1e-5)
_TAPS = 7
_BT = 8


def _ru(x, m):
    return (x + m - 1) // m * m


def _tcn_body(x_ref, w_ref, b_ref, recon_ref, sse_ref, *, prog):
    x = x_ref[...]
    Bt, T, F = x.shape

    def wp(meta):
        r0, rows, cols = meta
        return w_ref[r0:r0 + rows, :cols]

    def bp(meta):
        r0, cols = meta
        return b_ref[r0, :cols]

    def mm(h2, wm, bm):
        y = jnp.dot(h2, wp(wm), preferred_element_type=jnp.float32)
        return y + bp(bm)[None, :]

    def conv(h3, wm, bm, shifts):
        cin = h3.shape[-1]
        hb = h3.astype(jnp.bfloat16)
        pad = shifts[0]
        hp = (jnp.concatenate(
                  [jnp.zeros((Bt, pad, cin), jnp.bfloat16), hb], axis=1)
              if pad > 0 else hb)
        taps = [hp[:, pad - s:pad - s + T, :] for s in shifts]
        xcat = taps[0] if len(taps) == 1 else jnp.concatenate(taps, axis=-1)
        y = mm(xcat.reshape(Bt * T, len(shifts) * cin), wm, bm)
        return y.reshape(Bt, T, -1)

    h = x
    for op in prog:
        if op[0] == "b":
            _, shifts, m1, mb1, m2, mb2, md, mbd = op
            if md is not None:
                res = mm(h.astype(jnp.bfloat16).reshape(Bt * T, h.shape[-1]),
                         md, mbd).reshape(Bt, T, -1)
            else:
                res = h
            a = jnp.maximum(conv(h, m1, mb1, shifts), 0.0)
            a = jnp.maximum(conv(a, m2, mb2, shifts), 0.0)
            h = jnp.maximum(a + res, 0.0)
        else:
            _, wm, bm = op
            h = mm(h.astype(jnp.bfloat16).reshape(Bt * T, h.shape[-1]),
                   wm, bm).reshape(Bt, T, -1)

    recon_ref[...] = h
    d = (h - x).reshape(Bt * T, F)
    sse_ref[...] = jnp.sum(d * d).reshape(1, 1, 1)


def _pack_and_prog(enc_blocks, enc_head, dec_blocks, dec_head, T):
    warrs, barrs = [], []
    woff = [0]
    boff = [0]

    def addw(a):
        rows, cols = a.shape
        pr = _ru(rows, 16)
        pad = jnp.zeros((pr, 128), jnp.bfloat16)
        pad = pad.at[:rows, :cols].set(a.astype(jnp.bfloat16))
        meta = (woff[0], rows, cols)
        warrs.append(pad)
        woff[0] += pr
        return meta

    def addb(b):
        cols = b.shape[-1]
        row = jnp.zeros((8, 128), jnp.float32)
        row = row.at[0, :cols].set(b.reshape(-1))
        meta = (boff[0], cols)
        barrs.append(row)
        boff[0] += 8
        return meta

    prog = []

    def add_block(blk, dilation):
        w1, b1, w2, b2, wd, bd = blk
        K, cin, cout = w1.shape
        js = [j for j in range(K) if (K - 1 - j) * dilation < T]
        j0 = js[0]
        nt = len(js)
        shifts = tuple((K - 1 - j) * dilation for j in js)
        m1 = addw((w1[j0:] * _SCALE).reshape(nt * cin, cout))
        mb1 = addb(b1 * _SCALE)
        m2 = addw((w2[j0:] * _SCALE).reshape(nt * cout, cout))
        mb2 = addb(b2 * _SCALE)
        if wd is not None:
            md, mbd = addw(wd), addb(bd)
        else:
            md, mbd = None, None
        prog.append(("b", shifts, m1, mb1, m2, mb2, md, mbd))

    for i, blk in enumerate(enc_blocks):
        add_block(blk, 2 ** i)
    prog.append(("l", addw(enc_head[0]), addb(enc_head[1])))
    for i, blk in enumerate(dec_blocks):
        add_block(blk, 2 ** i)
    prog.append(("l", addw(dec_head[0]), addb(dec_head[1])))

    wpack = jnp.concatenate(warrs, axis=0)
    bpack = jnp.concatenate(barrs, axis=0)
    return wpack, bpack, tuple(prog)


def kernel(x,
           enc0_w1, enc0_b1, enc0_w2, enc0_b2, enc0_wd, enc0_bd,
           enc1_w1, enc1_b1, enc1_w2, enc1_b2,
           enc2_w1, enc2_b1, enc2_w2, enc2_b2,
           enc_out_w, enc_out_b,
           dec0_w1, dec0_b1, dec0_w2, dec0_b2, dec0_wd, dec0_bd,
           dec1_w1, dec1_b1, dec1_w2, dec1_b2,
           dec2_w1, dec2_b1, dec2_w2, dec2_b2,
           dec_out_w, dec_out_b):
    B, T, F = x.shape
    enc_blocks = [(enc0_w1, enc0_b1, enc0_w2, enc0_b2, enc0_wd, enc0_bd),
                  (enc1_w1, enc1_b1, enc1_w2, enc1_b2, None, None),
                  (enc2_w1, enc2_b1, enc2_w2, enc2_b2, None, None)]
    dec_blocks = [(dec0_w1, dec0_b1, dec0_w2, dec0_b2, dec0_wd, dec0_bd),
                  (dec1_w1, dec1_b1, dec1_w2, dec1_b2, None, None),
                  (dec2_w1, dec2_b1, dec2_w2, dec2_b2, None, None)]
    wpack, bpack, prog = _pack_and_prog(enc_blocks, (enc_out_w, enc_out_b),
                                        dec_blocks, (dec_out_w, dec_out_b), T)

    Bt = _BT if B % _BT == 0 and B // _BT >= 2 else max(
        bt for bt in range(1, B + 1) if B % bt == 0 and (B == 1 or B // bt >= 2))
    G = B // Bt

    import functools
    body = functools.partial(_tcn_body, prog=prog)

    flops = 0
    for op in prog:
        metas = ([op[2], op[4]] + ([op[6]] if op[6] is not None else [])
                 if op[0] == "b" else [op[1]])
        for (_, rows, cols) in metas:
            flops += 2 * B * T * rows * cols
    cost = pl.CostEstimate(flops=int(flops), transcendentals=0,
                           bytes_accessed=int(2 * wpack.size + 4 * bpack.size
                                              + 8 * B * T * F))

    recon, sse = pl.pallas_call(
        body,
        out_shape=(jax.ShapeDtypeStruct((B, T, F), jnp.float32),
                   jax.ShapeDtypeStruct((G, 1, 1), jnp.float32)),
        grid=(G,),
        in_specs=[pl.BlockSpec((Bt, T, F), lambda g: (g, 0, 0)),
                  pl.BlockSpec(wpack.shape, lambda g: (0, 0)),
                  pl.BlockSpec(bpack.shape, lambda g: (0, 0))],
        out_specs=(pl.BlockSpec((Bt, T, F), lambda g: (g, 0, 0)),
                   pl.BlockSpec((1, 1, 1), lambda g: (g, 0, 0))),
        compiler_params=pltpu.CompilerParams(
            dimension_semantics=("parallel",),
            vmem_limit_bytes=48 << 20),
        cost_estimate=cost,
    )(x, wpack, bpack)

    loss = jnp.sum(sse) / float(B * T * F)
    return loss, recon


# speedup vs baseline: 2.1632x; 1.9098x over previous
import functools
import math

import jax
import jax.numpy as jnp
from jax.experimental import pallas as pl
from jax.experimental.pallas import tpu as pltpu

_SCALE = 1.0 / math.sqrt(1.0 + 1e-5)
_HALO = 24


def _ru(x, m):
    return (x + m - 1) // m * m


def _tcn_body(xt_ref, w_ref, b_ref, recont_ref, sse_ref, halo_ref,
              *, prog, Tc, Bc):
    c = pl.program_id(1)
    x3 = xt_ref[...]
    Fdim = x3.shape[0]
    N = Tc * Bc
    xt = x3.reshape(Fdim, N)

    @pl.when(c == 0)
    def _zero_halo():
        halo_ref[...] = jnp.zeros_like(halo_ref)

    def wp(meta):
        r0, rows, cols = meta
        return w_ref[r0:r0 + rows, :cols]

    def bp(meta):
        r0, rows = meta
        return b_ref[r0:r0 + rows, 0:1]

    def mm(hb, wm, bm):
        y = jnp.dot(wp(wm), hb, preferred_element_type=jnp.float32)
        return y + bp(bm)

    def conv(h3, wm, bm, shifts, slot):
        cin = h3.shape[0]
        hb = h3.astype(jnp.bfloat16)
        H = shifts[0]
        if H > 0:
            hx = jnp.concatenate(
                [halo_ref[slot, :cin, :H * Bc], hb], axis=1)
            halo_ref[slot, :cin, :H * Bc] = hb[:, (Tc - H) * Bc:]
        else:
            hx = hb
        taps = [hx[:, (H - s) * Bc:(H - s) * Bc + N] for s in shifts]
        xcat = taps[0] if len(taps) == 1 else jnp.concatenate(taps, axis=0)
        return mm(xcat, wm, bm)

    h = xt
    slot = 0
    for op in prog:
        if op[0] == "b":
            _, shifts, m1, mb1, m2, mb2, md, mbd = op
            if md is not None:
                res = mm(h.astype(jnp.bfloat16), md, mbd)
            else:
                res = h
            a = jnp.maximum(conv(h, m1, mb1, shifts, slot), 0.0)
            a = jnp.maximum(conv(a, m2, mb2, shifts, slot + 1), 0.0)
            slot += 2
            h = jnp.maximum(a + res, 0.0)
        else:
            _, wm, bm = op
            h = mm(h.astype(jnp.bfloat16), wm, bm)

    recont_ref[...] = h.reshape(Fdim, Tc, Bc)
    d = h - xt
    sse_ref[...] = jnp.sum(d * d).reshape(1, 1, 1, 1)


def _pack_and_prog(enc_blocks, enc_head, dec_blocks, dec_head, T):
    warrs, barrs = [], []
    woff = [0]
    boff = [0]
    wcols = [128]

    def addw(a):
        rows, cols = a.shape
        wcols[0] = max(wcols[0], _ru(cols, 128))
        meta = (woff[0], rows, cols)
        warrs.append(a.astype(jnp.bfloat16))
        woff[0] += _ru(rows, 16)
        return meta

    def addb(b):
        cout = b.shape[-1]
        col = jnp.zeros((_ru(cout, 8), 128), jnp.float32)
        col = col.at[:cout, 0].set(b.reshape(-1))
        meta = (boff[0], cout)
        barrs.append(col)
        boff[0] += _ru(cout, 8)
        return meta

    prog = []

    def add_block(blk, dilation):
        w1, b1, w2, b2, wd, bd = blk
        K, cin, cout = w1.shape
        js = [j for j in range(K) if (K - 1 - j) * dilation < T]
        j0 = js[0]
        nt = len(js)
        shifts = tuple((K - 1 - j) * dilation for j in js)
        m1 = addw((w1[j0:] * _SCALE).reshape(nt * cin, cout).T)
        mb1 = addb(b1 * _SCALE)
        m2 = addw((w2[j0:] * _SCALE).reshape(nt * cout, cout).T)
        mb2 = addb(b2 * _SCALE)
        if wd is not None:
            md, mbd = addw(wd.T), addb(bd)
        else:
            md, mbd = None, None
        prog.append(("b", shifts, m1, mb1, m2, mb2, md, mbd))

    for i, blk in enumerate(enc_blocks):
        add_block(blk, 2 ** i)
    prog.append(("l", addw(enc_head[0].T), addb(enc_head[1])))
    for i, blk in enumerate(dec_blocks):
        add_block(blk, 2 ** i)
    prog.append(("l", addw(dec_head[0].T), addb(dec_head[1])))

    C = wcols[0]
    slabs = []
    for a in warrs:
        rows, cols = a.shape
        slabs.append(jnp.pad(a, ((0, _ru(rows, 16) - rows), (0, C - cols))))
    wpack = jnp.concatenate(slabs, axis=0)
    bpack = jnp.concatenate(barrs, axis=0)
    return wpack, bpack, tuple(prog)


def kernel(x,
           enc0_w1, enc0_b1, enc0_w2, enc0_b2, enc0_wd, enc0_bd,
           enc1_w1, enc1_b1, enc1_w2, enc1_b2,
           enc2_w1, enc2_b1, enc2_w2, enc2_b2,
           enc_out_w, enc_out_b,
           dec0_w1, dec0_b1, dec0_w2, dec0_b2, dec0_wd, dec0_bd,
           dec1_w1, dec1_b1, dec1_w2, dec1_b2,
           dec2_w1, dec2_b1, dec2_w2, dec2_b2,
           dec_out_w, dec_out_b):
    B, T, F = x.shape
    enc_blocks = [(enc0_w1, enc0_b1, enc0_w2, enc0_b2, enc0_wd, enc0_bd),
                  (enc1_w1, enc1_b1, enc1_w2, enc1_b2, None, None),
                  (enc2_w1, enc2_b1, enc2_w2, enc2_b2, None, None)]
    dec_blocks = [(dec0_w1, dec0_b1, dec0_w2, dec0_b2, dec0_wd, dec0_bd),
                  (dec1_w1, dec1_b1, dec1_w2, dec1_b2, None, None),
                  (dec2_w1, dec2_b1, dec2_w2, dec2_b2, None, None)]
    wpack, bpack, prog = _pack_and_prog(enc_blocks, (enc_out_w, enc_out_b),
                                        dec_blocks, (dec_out_w, dec_out_b), T)

    Bc = 128 if B % 128 == 0 else B
    Gb = B // Bc
    Tc = T
    for cand in (64, 32):
        if T % cand == 0 and T // cand >= 1 and cand > _HALO:
            Tc = cand
            break
    Gc = T // Tc
    n_conv = sum(2 for op in prog if op[0] == "b")

    xt = x.transpose(2, 1, 0)

    body = functools.partial(_tcn_body, prog=prog, Tc=Tc, Bc=Bc)

    flops = 0
    for op in prog:
        metas = ([op[2], op[4]] + ([op[6]] if op[6] is not None else [])
                 if op[0] == "b" else [op[1]])
        for (_, rows, cols) in metas:
            flops += 2 * B * T * rows * cols
    cost = pl.CostEstimate(flops=int(flops), transcendentals=0,
                           bytes_accessed=int(2 * wpack.size + 4 * bpack.size
                                              + 8 * B * T * F))

    recont, sse = pl.pallas_call(
        body,
        out_shape=(jax.ShapeDtypeStruct((F, T, B), jnp.float32),
                   jax.ShapeDtypeStruct((Gb, Gc, 1, 1), jnp.float32)),
        grid=(Gb, Gc),
        in_specs=[pl.BlockSpec((F, Tc, Bc), lambda g, c: (0, c, g)),
                  pl.BlockSpec(wpack.shape, lambda g, c: (0, 0)),
                  pl.BlockSpec(bpack.shape, lambda g, c: (0, 0))],
        out_specs=(pl.BlockSpec((F, Tc, Bc), lambda g, c: (0, c, g)),
                   pl.BlockSpec((1, 1, 1, 1), lambda g, c: (g, c, 0, 0))),
        scratch_shapes=[pltpu.VMEM((n_conv, 128, _HALO * Bc), jnp.bfloat16)],
        compiler_params=pltpu.CompilerParams(
            dimension_semantics=("parallel", "arbitrary"),
            vmem_limit_bytes=48 << 20),
        cost_estimate=cost,
    )(xt, wpack, bpack)

    recon = recont.transpose(2, 1, 0)
    loss = jnp.sum(sse) / float(B * T * F)
    return loss, recon
